# Initial kernel scaffold; baseline (speedup 1.0000x reference)
import jax
import jax.numpy as jnp
from jax.experimental import pallas as pl


def kernel(p1, p2, z1, z2, anchor_features, corrs1, corrs2, labels1, labels2):
    raise NotImplementedError("write your pallas kernel here")


# speedup vs baseline: 11.0687x; 11.0687x over previous
import jax
import jax.numpy as jnp
from jax.experimental import pallas as pl
from jax.experimental.pallas import tpu as pltpu

_NUM_CLASSES = 20
_EPS = 1e-12


def _tc_body(p1_ref, p2_ref, l1_ref, l2_ref, a_ref,
             loss_ref, sl1_ref, sl2_ref, si1_ref, si2_ref,
             s1_acc, s2_acc, c1_acc, c2_acc):
    i = pl.program_id(0)
    B = p1_ref.shape[0]

    @pl.when(i == 0)
    def _init():
        s1_acc[...] = jnp.zeros_like(s1_acc)
        s2_acc[...] = jnp.zeros_like(s2_acc)
        c1_acc[...] = jnp.zeros_like(c1_acc)
        c2_acc[...] = jnp.zeros_like(c2_acc)

    for p_ref, l_ref, s_acc, c_acc in (
        (p1_ref, l1_ref, s1_acc, c1_acc),
        (p2_ref, l2_ref, s2_acc, c2_acc),
    ):
        p = p_ref[...]
        ss = jnp.sum(p * p, axis=1, keepdims=True)
        rinv = 1.0 / jnp.maximum(jnp.sqrt(ss), _EPS)
        pn = p * rinv
        lbl = l_ref[0]
        lblb = jnp.broadcast_to(lbl, (128, B))
        ohT = (lblb == jax.lax.broadcasted_iota(jnp.int32, (128, B), 0)
               ).astype(jnp.float32)
        s_acc[...] += jax.lax.dot_general(
            ohT, pn, (((1,), (0,)), ((), ())),
            preferred_element_type=jnp.float32)
        c_acc[...] += jnp.sum(ohT, axis=1, keepdims=True)

    @pl.when(i == pl.num_programs(0) - 1)
    def _fin():
        a = a_ref[...]
        an = a / jnp.maximum(
            jnp.sqrt(jnp.sum(a * a, axis=1, keepdims=True)), _EPS)

        def branch(s_acc, c_acc):
            s20 = s_acc[0:_NUM_CLASSES, :]
            cnt = c_acc[0:_NUM_CLASSES, :]
            dots = jnp.sum(s20 * an, axis=1, keepdims=True)
            sl = (cnt - dots) * 0.25
            mean = sl / jnp.maximum(cnt, 1.0)
            present = (cnt > 0).astype(jnp.float32)
            bal = (jnp.sum(mean * present, axis=0, keepdims=True) /
                   jnp.maximum(jnp.sum(present, axis=0, keepdims=True), 1.0))
            return sl, cnt, bal

        sl1, c1, b1 = branch(s1_acc, c1_acc)
        sl2, c2, b2 = branch(s2_acc, c2_acc)
        loss_ref[...] = b1 + b2
        sl1_ref[...] = sl1
        si1_ref[...] = c1
        sl2_ref[...] = sl2
        si2_ref[...] = c2


def kernel(p1, p2, z1, z2, anchor_features, corrs1, corrs2, labels1, labels2):
    N, D = p1.shape
    B = 2048
    G = N // B
    l1r = labels1.reshape(G, 1, B)
    l2r = labels2.reshape(G, 1, B)
    f32 = jnp.float32
    outs = pl.pallas_call(
        _tc_body,
        grid=(G,),
        in_specs=[
            pl.BlockSpec((B, D), lambda i: (i, 0)),
            pl.BlockSpec((B, D), lambda i: (i, 0)),
            pl.BlockSpec((1, 1, B), lambda i: (i, 0, 0)),
            pl.BlockSpec((1, 1, B), lambda i: (i, 0, 0)),
            pl.BlockSpec((_NUM_CLASSES, D), lambda i: (0, 0)),
        ],
        out_specs=[
            pl.BlockSpec((1, 1), lambda i: (0, 0)),
            pl.BlockSpec((_NUM_CLASSES, 1), lambda i: (0, 0)),
            pl.BlockSpec((_NUM_CLASSES, 1), lambda i: (0, 0)),
            pl.BlockSpec((_NUM_CLASSES, 1), lambda i: (0, 0)),
            pl.BlockSpec((_NUM_CLASSES, 1), lambda i: (0, 0)),
        ],
        out_shape=[
            jax.ShapeDtypeStruct((1, 1), f32),
            jax.ShapeDtypeStruct((_NUM_CLASSES, 1), f32),
            jax.ShapeDtypeStruct((_NUM_CLASSES, 1), f32),
            jax.ShapeDtypeStruct((_NUM_CLASSES, 1), f32),
            jax.ShapeDtypeStruct((_NUM_CLASSES, 1), f32),
        ],
        scratch_shapes=[
            pltpu.VMEM((128, 128), f32),
            pltpu.VMEM((128, 128), f32),
            pltpu.VMEM((128, 1), f32),
            pltpu.VMEM((128, 1), f32),
        ],
        compiler_params=pltpu.CompilerParams(
            dimension_semantics=("arbitrary",),
        ),
    )(p1, p2, l1r, l2r, anchor_features)
    loss, sl1, sl2, si1, si2 = outs
    return (loss[0, 0], sl1[:, 0], sl2[:, 0], si1[:, 0], si2[:, 0])
